# baseline (speedup 1.0000x reference)
import functools

import jax
import jax.numpy as jnp
from jax.experimental import pallas as pl
from jax.experimental.pallas import tpu as pltpu
from jax.experimental.pallas import tpu_sc as plsc

C = 384
CLSN = 150
NH = 4
HD = C // NH
HP = 32
WP = 32
N = HP * WP
KNN = 9
G = NH * CLSN
EPS = 1e-5
SCALE = HD ** -0.5
DPAD = 640
NWORK = 32


def _ln_rows(x, g, b):
    m = jnp.mean(x, axis=1, keepdims=True)
    v = jnp.mean((x - m) ** 2, axis=1, keepdims=True)
    return (x - m) * jax.lax.rsqrt(v + EPS) * g + b


def _lng_block(x, w, b):
    m = jnp.mean(x)
    v = jnp.mean((x - m) ** 2)
    return (x - m) * jax.lax.rsqrt(v + EPS) * w + b


def _dot(a, b):
    return jax.lax.dot_general(a, b, (((1,), (0,)), ((), ())),
                               preferred_element_type=jnp.float32)


def _dot_tb(a, b):
    return jax.lax.dot_general(a, b, (((1,), (1,)), ((), ())),
                               preferred_element_type=jnp.float32)


def _gelu(x):
    return jax.nn.gelu(x, approximate=True)


def _ka_body(xc_ref, xp_ref, ncls_g, ncls_b, qw, qb, kvw, kvb, nxw, nxb,
             projw, projb, w9, ppb, ppg, ppbb,
             ncls_g2, ncls_b2, qw2, qb2, kvw2, kvb2, nxw2, nxb2,
             out_cls_ref, xp2_ref, attn_ref, vv_ref):
    xc = xc_ref[0]
    xp = xp_ref[0]
    xl = _ln_rows(xc, ncls_g[...], ncls_b[...])
    q = _dot_tb(xl, qw[...]) + qb[...]
    xn = _lng_block(xp, nxw[...], nxb[...])
    kv = _dot(kvw[...], xn) + kvb[...]
    outs = []
    for h in range(NH):
        qh = q[:, h * HD:(h + 1) * HD]
        kh = kv[h * HD:(h + 1) * HD, :]
        vh = kv[C + h * HD:C + (h + 1) * HD, :]
        lg = _dot(qh, kh) * SCALE
        lg = lg - jnp.max(lg, axis=1, keepdims=True)
        e = jnp.exp(lg)
        p = e / jnp.sum(e, axis=1, keepdims=True)
        outs.append(_dot_tb(p, vh))
    oc = jnp.concatenate(outs, axis=1)
    out_cls = xc + _dot_tb(oc, projw[...]) + projb[...]
    out_cls_ref[0] = out_cls

    op = _dot(out_cls, xp)
    col = jax.lax.broadcasted_iota(jnp.int32, (1, N), 1) % WP
    row = jax.lax.broadcasted_iota(jnp.int32, (1, N), 1) // WP
    acc = jnp.zeros((C, N), jnp.float32)
    for ky in range(3):
        for kx in range(3):
            off = (ky - 1) * WP + (kx - 1)
            if off > 0:
                sh = jnp.concatenate(
                    [op[:, off:], jnp.zeros((CLSN, off), jnp.float32)], axis=1)
            elif off < 0:
                sh = jnp.concatenate(
                    [jnp.zeros((CLSN, -off), jnp.float32), op[:, :N + off]],
                    axis=1)
            else:
                sh = op
            mask = ((col + (kx - 1) >= 0) & (col + (kx - 1) < WP) &
                    (row + (ky - 1) >= 0) & (row + (ky - 1) < HP))
            sh = jnp.where(mask, sh, 0.0)
            acc = acc + _dot(w9[3 * ky + kx], sh)
    op2 = (acc + ppb[...]) * ppg[...] + ppbb[...]
    xp2 = xp + _gelu(op2)
    xp2_ref[0] = xp2

    xn2 = _lng_block(xp2, nxw2[...], nxb2[...])
    q2 = _dot(qw2[...], xn2) + qb2[...]
    clsn = _ln_rows(out_cls, ncls_g2[...], ncls_b2[...])
    kv2 = _dot_tb(clsn, kvw2[...]) + kvb2[...]
    kk = kv2[:, :C]
    vv_ref[0] = kv2[:, C:]
    blocks = []
    for h in range(NH):
        kh = kk[:, h * HD:(h + 1) * HD]
        qh = q2[h * HD:(h + 1) * HD, :]
        blocks.append(_dot(kh, qh) * SCALE)
    attn_ref[0] = jnp.concatenate(blocks, axis=0)


def _kb_body(x_ref, fc1w, fc1b, fc1g, fc1bb, x1_ref, fpad_ref, idx_ref):
    x = x_ref[0]
    x1 = _dot(fc1w[...], x) + fc1b[...]
    x1 = x1 * fc1g[...] + fc1bb[...]
    x1_ref[0] = x1

    f = x1.T
    fpad_ref[0, :, :G] = f
    fpad_ref[0, :, G:] = jnp.zeros((N, DPAD - G), jnp.float32)

    gram = _dot_tb(f, f)
    sq_row = jnp.sum(x1 * x1, axis=0, keepdims=True)
    dist = sq_row - 2.0 * gram

    gbase = pl.program_id(0) * N
    iotaf = jax.lax.broadcasted_iota(jnp.int32, (N, N), 1).astype(jnp.float32)
    for k in range(KNN):
        vmin = jnp.min(dist, axis=1, keepdims=True)
        idxf = jnp.min(jnp.where(dist <= vmin, iotaf, jnp.float32(2.0 * N)),
                       axis=1, keepdims=True)
        idx_ref[0, :, k:k + 1] = idxf.astype(jnp.int32) + gbase
        if k < KNN - 1:
            dist = jnp.where(iotaf == idxf, jnp.inf, dist)


def _sc_gather_max(table, idx):
    rows, d = table.shape
    npw = rows // NWORK
    mesh = plsc.VectorSubcoreMesh(core_axis_name="c", subcore_axis_name="s")

    nvr = (G + 15) // 16
    half = npw // 2
    hidx = KNN * half

    @functools.partial(
        pl.kernel, mesh=mesh,
        out_type=jax.ShapeDtypeStruct((rows, d), jnp.float32),
        scratch_types=[
            pltpu.VMEM((2 * hidx,), jnp.int32),
            pltpu.VMEM((half, d), jnp.float32),
            pltpu.VMEM((half, d), jnp.float32),
            pltpu.VMEM((half, d), jnp.float32),
            pltpu.VMEM((half, d), jnp.float32),
            pltpu.VMEM((half, d), jnp.float32),
            pltpu.SemaphoreType.DMA,
            pltpu.SemaphoreType.DMA,
            pltpu.SemaphoreType.DMA,
            pltpu.SemaphoreType.DMA,
            pltpu.SemaphoreType.DMA,
        ],
    )
    def run(table_hbm, idx_hbm, out_hbm, idxv, accv, r0, r1, r2, r3,
            sema, s0, s1, s2, s3):
        wid = jax.lax.axis_index("s") * 2 + jax.lax.axis_index("c")
        rbuf = (r0, r1, r2, r3)
        rsem = (s0, s1, s2, s3)
        pltpu.sync_copy(idx_hbm.at[wid], idxv)

        def gather(h, k, dst, sem):
            isl = idxv.at[pl.ds(h * hidx + k * half, half)]
            return pltpu.async_copy(table_hbm.at[isl], dst, sem)

        for h in range(2):
            cp_acc = gather(h, 0, accv, sema)
            cps = {1: gather(h, 1, r0, s0), 2: gather(h, 2, r1, s1)}
            cp_acc.wait()
            for p in range(4):
                if p < 3:
                    ba, bb = (2 * (p + 1)) % 4, (2 * (p + 1) + 1) % 4
                    cps[2 * p + 3] = gather(h, 2 * p + 3, rbuf[ba], rsem[ba])
                    cps[2 * p + 4] = gather(h, 2 * p + 4, rbuf[bb], rsem[bb])
                cps[2 * p + 1].wait()
                cps[2 * p + 2].wait()
                ra = rbuf[(2 * p) % 4]
                rb = rbuf[(2 * p + 1) % 4]

                def body(i, _, _ra=ra, _rb=rb):
                    for j in range(nvr):
                        sl = pl.ds(j * 16, 16)
                        accv[i, sl] = jnp.maximum(
                            accv[i, sl], jnp.maximum(_ra[i, sl], _rb[i, sl]))
                    return 0

                jax.lax.fori_loop(0, half, body, 0)
            pltpu.sync_copy(accv,
                            out_hbm.at[pl.ds(wid * npw + h * half, half)])

    return run(table, idx)


def _kc_body(x_ref, x1_ref, mt_ref, vv_ref, cls_ref, xp2_ref,
             wfm, wm, nnb, nng, nnbb, fc2w, fc2b, fc2g, fc2bb,
             projw, projb, normg, normb, m1w, m1b, m2w, m2b,
             cls_out_ref, patch_out_ref):
    x = x_ref[0]
    x1 = x1_ref[0]
    mt = mt_ref[0]
    ys = []
    for g in range(NH):
        xg = x1[g * CLSN:(g + 1) * CLSN, :]
        mtg = mt[:, g * CLSN:(g + 1) * CLSN]
        ys.append(_dot(wfm[g], xg) + _dot_tb(wm[g], mtg))
    y = jnp.concatenate(ys, axis=0) + nnb[...]
    y = _gelu(y * nng[...] + nnbb[...])
    gout = _dot(fc2w[...], y) + fc2b[...]
    gout = gout * fc2g[...] + fc2bb[...] + x

    vv = vv_ref[0]
    vvt = vv.T
    outs = []
    for h in range(NH):
        blk = gout[h * CLSN:(h + 1) * CLSN, :]
        blk = blk - jnp.max(blk, axis=0, keepdims=True)
        e = jnp.exp(blk)
        p = e / jnp.sum(e, axis=0, keepdims=True)
        vh = vvt[h * HD:(h + 1) * HD, :]
        outs.append(_dot(vh, p))
    o = jnp.concatenate(outs, axis=0)
    patch_out_ref[0] = xp2_ref[0] + _dot(projw[...], o) + projb[...]

    xc = cls_ref[0]
    hl = _ln_rows(xc, normg[...], normb[...])
    h1 = _gelu(_dot_tb(hl, m1w[...]) + m1b[...])
    h2 = _dot_tb(h1, m2w[...]) + m2b[...]
    cls_out_ref[0] = xc + h2


def _bspec(shape):
    nz = (0,) * len(shape)
    return pl.BlockSpec(shape, lambda b, _z=nz: _z)


def _bspecB(shape):
    nz = (0,) * len(shape)
    return pl.BlockSpec((1,) + shape, lambda b, _z=nz: (b,) + _z)


def _call(body, batch, ins, in_shapes, out_shapes, out_dtypes=None):
    in_specs = [(_bspecB(s) if bt else _bspec(s)) for (_, bt), s in
                zip(ins, in_shapes)]
    out_specs = [_bspecB(s) for s in out_shapes]
    if out_dtypes is None:
        out_dtypes = [jnp.float32] * len(out_shapes)
    out_shape = [jax.ShapeDtypeStruct((batch,) + s, dt)
                 for s, dt in zip(out_shapes, out_dtypes)]
    return pl.pallas_call(
        body, grid=(batch,), in_specs=in_specs, out_specs=out_specs,
        out_shape=out_shape,
    )(*[a for a, _ in ins])


def kernel(x_cls, x_patch, params):
    batch = x_cls.shape[0]
    f32 = jnp.float32
    p1 = params['s2f']
    p2 = params['f2s']
    pg = p2['grapher']
    xp = x_patch.reshape(batch, C, N)

    r2 = lambda a: a.reshape(-1, 1).astype(f32)
    r1 = lambda a: a.reshape(1, -1).astype(f32)

    w9 = p1['pp_w'].transpose(2, 3, 0, 1).reshape(9, C, CLSN)
    ka_ins = [
        (x_cls, True), (xp, True),
        (r1(p1['ncls_g']), False), (r1(p1['ncls_b']), False),
        (p1['q_w'], False), (r1(p1['q_b']), False),
        (p1['kv_w'], False), (r2(p1['kv_b']), False),
        (r2(p1['nx_w']), False), (r2(p1['nx_b']), False),
        (p1['proj_w'], False), (r1(p1['proj_b']), False),
        (w9, False), (r2(p1['pp_b']), False),
        (r2(p1['pp_bn_g']), False), (r2(p1['pp_bn_b']), False),
        (r1(p2['ncls_g']), False), (r1(p2['ncls_b']), False),
        (p2['q_w'], False), (r2(p2['q_b']), False),
        (p2['kv_w'], False), (r1(p2['kv_b']), False),
        (r2(p2['nx_w']), False), (r2(p2['nx_b']), False),
    ]
    ka_shapes = [(CLSN, C), (C, N), (1, C), (1, C), (C, C), (1, C),
                 (2 * C, C), (2 * C, 1), (C, 1), (C, 1), (C, C), (1, C),
                 (9, C, CLSN), (C, 1), (C, 1), (C, 1),
                 (1, C), (1, C), (C, C), (C, 1), (2 * C, C), (1, 2 * C),
                 (C, 1), (C, 1)]
    out_cls, xp2, attn_pre, vv = _call(
        _ka_body, batch, ka_ins, ka_shapes,
        [(CLSN, C), (C, N), (G, N), (CLSN, C)])

    kb_ins = [
        (attn_pre, True),
        (pg['fc1_w'], False), (r2(pg['fc1_b']), False),
        (r2(pg['fc1_bn_g']), False), (r2(pg['fc1_bn_b']), False),
    ]
    kb_shapes = [(G, N), (G, G), (G, 1), (G, 1), (G, 1)]

    x1_l, maxnt_l = [], []
    npw = N // NWORK
    for b in range(batch):
        kb_ins_b = [(attn_pre[b:b + 1], True)] + kb_ins[1:]
        x1b_b, fpad_b, idx_b = _call(_kb_body, 1, kb_ins_b, kb_shapes,
                                     [(G, N), (N, DPAD), (N, KNN)],
                                     [jnp.float32, jnp.float32, jnp.int32])
        idx_sc = (idx_b[0].transpose(1, 0)
                  .reshape(KNN, NWORK, 2, npw // 2)
                  .transpose(1, 2, 0, 3).reshape(NWORK, KNN * npw))
        maxnt_l.append(_sc_gather_max(fpad_b[0], idx_sc))
        x1_l.append(x1b_b)
    x1b = jnp.concatenate(x1_l, axis=0)
    maxnt = jnp.stack(maxnt_l, axis=0)

    wf = pg['nn_w'][:, :, 0::2]
    wm = pg['nn_w'][:, :, 1::2]
    wfm = wf - wm
    gpg = 2 * G // NH
    kc_ins = [
        (attn_pre, True), (x1b, True), (maxnt, True),
        (vv, True), (out_cls, True), (xp2, True),
        (wfm, False), (wm, False),
        (r2(pg['nn_b']), False), (r2(pg['nn_bn_g']), False),
        (r2(pg['nn_bn_b']), False),
        (pg['fc2_w'], False), (r2(pg['fc2_b']), False),
        (r2(pg['fc2_bn_g']), False), (r2(pg['fc2_bn_b']), False),
        (p2['proj_w'], False), (r2(p2['proj_b']), False),
        (r1(params['norm_g']), False), (r1(params['norm_b']), False),
        (params['mlp_fc1_w'], False), (r1(params['mlp_fc1_b']), False),
        (params['mlp_fc2_w'], False), (r1(params['mlp_fc2_b']), False),
    ]
    kc_shapes = [(G, N), (G, N), (N, DPAD), (CLSN, C), (CLSN, C), (C, N),
                 (NH, gpg, CLSN), (NH, gpg, CLSN),
                 (2 * G, 1), (2 * G, 1), (2 * G, 1),
                 (G, 2 * G), (G, 1), (G, 1), (G, 1),
                 (C, C), (C, 1), (1, C), (1, C),
                 (4 * C, C), (1, 4 * C), (C, 4 * C), (1, C)]
    cls_l, patch_l = [], []
    for b in range(batch):
        kc_ins_b = [(a[b:b + 1], True) if bt else (a, False)
                    for a, bt in kc_ins]
        co, po = _call(_kc_body, 1, kc_ins_b, kc_shapes,
                       [(CLSN, C), (C, N)])
        cls_l.append(co)
        patch_l.append(po)
    cls_out = jnp.concatenate(cls_l, axis=0)
    patch_out = jnp.concatenate(patch_l, axis=0)
    return cls_out, patch_out.reshape(batch, C, HP, WP)


# speedup vs baseline: 1.1396x; 1.1396x over previous
import functools

import jax
import jax.numpy as jnp
from jax.experimental import pallas as pl
from jax.experimental.pallas import tpu as pltpu
from jax.experimental.pallas import tpu_sc as plsc

C = 384
CLSN = 150
NH = 4
HD = C // NH
HP = 32
WP = 32
N = HP * WP
KNN = 9
G = NH * CLSN
EPS = 1e-5
SCALE = HD ** -0.5
DPAD = 640
NWORK = 32


def _ln_rows(x, g, b):
    m = jnp.mean(x, axis=1, keepdims=True)
    v = jnp.mean((x - m) ** 2, axis=1, keepdims=True)
    return (x - m) * jax.lax.rsqrt(v + EPS) * g + b


def _lng_block(x, w, b):
    m = jnp.mean(x)
    v = jnp.mean((x - m) ** 2)
    return (x - m) * jax.lax.rsqrt(v + EPS) * w + b


def _dot(a, b):
    return jax.lax.dot_general(a, b, (((1,), (0,)), ((), ())),
                               preferred_element_type=jnp.float32)


def _dot_tb(a, b):
    return jax.lax.dot_general(a, b, (((1,), (1,)), ((), ())),
                               preferred_element_type=jnp.float32)


def _gelu(x):
    return jax.nn.gelu(x, approximate=True)


def _ka_body(xc_ref, xp_ref, ncls_g, ncls_b, qw, qb, kvw, kvb, nxw, nxb,
             projw, projb, w9, ppb, ppg, ppbb,
             ncls_g2, ncls_b2, qw2, qb2, kvw2, kvb2, nxw2, nxb2,
             out_cls_ref, xp2_ref, attn_ref, vv_ref):
    xc = xc_ref[0]
    xp = xp_ref[0]
    xl = _ln_rows(xc, ncls_g[...], ncls_b[...])
    q = _dot_tb(xl, qw[...]) + qb[...]
    xn = _lng_block(xp, nxw[...], nxb[...])
    kv = _dot(kvw[...], xn) + kvb[...]
    outs = []
    for h in range(NH):
        qh = q[:, h * HD:(h + 1) * HD]
        kh = kv[h * HD:(h + 1) * HD, :]
        vh = kv[C + h * HD:C + (h + 1) * HD, :]
        lg = _dot(qh, kh) * SCALE
        lg = lg - jnp.max(lg, axis=1, keepdims=True)
        e = jnp.exp(lg)
        p = e / jnp.sum(e, axis=1, keepdims=True)
        outs.append(_dot_tb(p, vh))
    oc = jnp.concatenate(outs, axis=1)
    out_cls = xc + _dot_tb(oc, projw[...]) + projb[...]
    out_cls_ref[0] = out_cls

    op = _dot(out_cls, xp)
    col = jax.lax.broadcasted_iota(jnp.int32, (1, N), 1) % WP
    row = jax.lax.broadcasted_iota(jnp.int32, (1, N), 1) // WP
    acc = jnp.zeros((C, N), jnp.float32)
    for ky in range(3):
        for kx in range(3):
            off = (ky - 1) * WP + (kx - 1)
            if off > 0:
                sh = jnp.concatenate(
                    [op[:, off:], jnp.zeros((CLSN, off), jnp.float32)], axis=1)
            elif off < 0:
                sh = jnp.concatenate(
                    [jnp.zeros((CLSN, -off), jnp.float32), op[:, :N + off]],
                    axis=1)
            else:
                sh = op
            mask = ((col + (kx - 1) >= 0) & (col + (kx - 1) < WP) &
                    (row + (ky - 1) >= 0) & (row + (ky - 1) < HP))
            sh = jnp.where(mask, sh, 0.0)
            acc = acc + _dot(w9[3 * ky + kx], sh)
    op2 = (acc + ppb[...]) * ppg[...] + ppbb[...]
    xp2 = xp + _gelu(op2)
    xp2_ref[0] = xp2

    xn2 = _lng_block(xp2, nxw2[...], nxb2[...])
    q2 = _dot(qw2[...], xn2) + qb2[...]
    clsn = _ln_rows(out_cls, ncls_g2[...], ncls_b2[...])
    kv2 = _dot_tb(clsn, kvw2[...]) + kvb2[...]
    kk = kv2[:, :C]
    vv_ref[0] = kv2[:, C:]
    blocks = []
    for h in range(NH):
        kh = kk[:, h * HD:(h + 1) * HD]
        qh = q2[h * HD:(h + 1) * HD, :]
        blocks.append(_dot(kh, qh) * SCALE)
    attn_ref[0] = jnp.concatenate(blocks, axis=0)


def _kb_body(x_ref, fc1w, fc1b, fc1g, fc1bb, x1_ref, fpad_ref, idx_ref):
    x = x_ref[0]
    x1 = _dot(fc1w[...], x) + fc1b[...]
    x1 = x1 * fc1g[...] + fc1bb[...]
    x1_ref[0] = x1

    f = x1.T
    fpad_ref[0, :, :G] = f
    fpad_ref[0, :, G:] = jnp.zeros((N, DPAD - G), jnp.float32)

    gram = _dot_tb(f, f)
    sq_row = jnp.sum(x1 * x1, axis=0, keepdims=True)
    dist = sq_row - 2.0 * gram

    gbase = pl.program_id(0) * N
    iotaf = jax.lax.broadcasted_iota(jnp.int32, (N, N), 1).astype(jnp.float32)
    for k in range(KNN):
        vmin = jnp.min(dist, axis=1, keepdims=True)
        idxf = jnp.min(jnp.where(dist <= vmin, iotaf, jnp.float32(2.0 * N)),
                       axis=1, keepdims=True)
        idx_ref[0, :, k:k + 1] = idxf.astype(jnp.int32) + gbase
        if k < KNN - 1:
            dist = jnp.where(iotaf == idxf, jnp.inf, dist)


def _sc_gather_max(table, idx):
    rows, d = table.shape
    npw = rows // NWORK
    mesh = plsc.VectorSubcoreMesh(core_axis_name="c", subcore_axis_name="s")

    nvr = (G + 15) // 16
    half = npw // 2
    hidx = KNN * half

    @functools.partial(
        pl.kernel, mesh=mesh,
        out_type=jax.ShapeDtypeStruct((rows, d), jnp.float32),
        scratch_types=[
            pltpu.VMEM((2 * hidx,), jnp.int32),
            pltpu.VMEM((half, d), jnp.float32),
            pltpu.VMEM((half, d), jnp.float32),
            pltpu.VMEM((half, d), jnp.float32),
            pltpu.VMEM((half, d), jnp.float32),
            pltpu.VMEM((half, d), jnp.float32),
            pltpu.SemaphoreType.DMA,
            pltpu.SemaphoreType.DMA,
            pltpu.SemaphoreType.DMA,
            pltpu.SemaphoreType.DMA,
            pltpu.SemaphoreType.DMA,
        ],
    )
    def run(table_hbm, idx_hbm, out_hbm, idxv, accv, r0, r1, r2, r3,
            sema, s0, s1, s2, s3):
        wid = jax.lax.axis_index("s") * 2 + jax.lax.axis_index("c")
        rbuf = (r0, r1, r2, r3)
        rsem = (s0, s1, s2, s3)
        pltpu.sync_copy(idx_hbm.at[wid], idxv)

        def gather(h, k, dst, sem):
            isl = idxv.at[pl.ds(h * hidx + k * half, half)]
            return pltpu.async_copy(table_hbm.at[isl], dst, sem)

        for h in range(2):
            cp_acc = gather(h, 0, accv, sema)
            cps = {1: gather(h, 1, r0, s0), 2: gather(h, 2, r1, s1)}
            cp_acc.wait()
            for p in range(4):
                if p < 3:
                    ba, bb = (2 * (p + 1)) % 4, (2 * (p + 1) + 1) % 4
                    cps[2 * p + 3] = gather(h, 2 * p + 3, rbuf[ba], rsem[ba])
                    cps[2 * p + 4] = gather(h, 2 * p + 4, rbuf[bb], rsem[bb])
                cps[2 * p + 1].wait()
                cps[2 * p + 2].wait()
                ra = rbuf[(2 * p) % 4]
                rb = rbuf[(2 * p + 1) % 4]

                def body(i, _, _ra=ra, _rb=rb):
                    for j in range(nvr):
                        sl = pl.ds(j * 16, 16)
                        accv[i, sl] = jnp.maximum(
                            accv[i, sl], jnp.maximum(_ra[i, sl], _rb[i, sl]))
                    return 0

                jax.lax.fori_loop(0, half, body, 0)
            pltpu.sync_copy(accv,
                            out_hbm.at[pl.ds(wid * npw + h * half, half)])

    return run(table, idx)


def _kc_body(x_ref, x1_ref, mt_ref, vv_ref, cls_ref, xp2_ref,
             wfm, wm, nnb, nng, nnbb, fc2w, fc2b, fc2g, fc2bb,
             projw, projb, normg, normb, m1w, m1b, m2w, m2b,
             cls_out_ref, patch_out_ref):
    x = x_ref[0]
    x1 = x1_ref[0]
    mt = mt_ref[0]
    ys = []
    for g in range(NH):
        xg = x1[g * CLSN:(g + 1) * CLSN, :]
        mtg = mt[:, g * CLSN:(g + 1) * CLSN]
        ys.append(_dot(wfm[g], xg) + _dot_tb(wm[g], mtg))
    y = jnp.concatenate(ys, axis=0) + nnb[...]
    y = _gelu(y * nng[...] + nnbb[...])
    gout = _dot(fc2w[...], y) + fc2b[...]
    gout = gout * fc2g[...] + fc2bb[...] + x

    vv = vv_ref[0]
    vvt = vv.T
    outs = []
    for h in range(NH):
        blk = gout[h * CLSN:(h + 1) * CLSN, :]
        blk = blk - jnp.max(blk, axis=0, keepdims=True)
        e = jnp.exp(blk)
        p = e / jnp.sum(e, axis=0, keepdims=True)
        vh = vvt[h * HD:(h + 1) * HD, :]
        outs.append(_dot(vh, p))
    o = jnp.concatenate(outs, axis=0)
    patch_out_ref[0] = xp2_ref[0] + _dot(projw[...], o) + projb[...]

    xc = cls_ref[0]
    hl = _ln_rows(xc, normg[...], normb[...])
    h1 = _gelu(_dot_tb(hl, m1w[...]) + m1b[...])
    h2 = _dot_tb(h1, m2w[...]) + m2b[...]
    cls_out_ref[0] = xc + h2


def _bspec(shape):
    nz = (0,) * len(shape)
    return pl.BlockSpec(shape, lambda b, _z=nz: _z)


def _bspecB(shape):
    nz = (0,) * len(shape)
    return pl.BlockSpec((1,) + shape, lambda b, _z=nz: (b,) + _z)


def _call(body, batch, ins, in_shapes, out_shapes, out_dtypes=None):
    in_specs = [(_bspecB(s) if bt else _bspec(s)) for (_, bt), s in
                zip(ins, in_shapes)]
    out_specs = [_bspecB(s) for s in out_shapes]
    if out_dtypes is None:
        out_dtypes = [jnp.float32] * len(out_shapes)
    out_shape = [jax.ShapeDtypeStruct((batch,) + s, dt)
                 for s, dt in zip(out_shapes, out_dtypes)]
    return pl.pallas_call(
        body, grid=(batch,), in_specs=in_specs, out_specs=out_specs,
        out_shape=out_shape,
    )(*[a for a, _ in ins])


def kernel(x_cls, x_patch, params):
    batch = x_cls.shape[0]
    f32 = jnp.float32
    p1 = params['s2f']
    p2 = params['f2s']
    pg = p2['grapher']
    xp = x_patch.reshape(batch, C, N)

    r2 = lambda a: a.reshape(-1, 1).astype(f32)
    r1 = lambda a: a.reshape(1, -1).astype(f32)

    w9 = p1['pp_w'].transpose(2, 3, 0, 1).reshape(9, C, CLSN)
    ka_ins = [
        (x_cls, True), (xp, True),
        (r1(p1['ncls_g']), False), (r1(p1['ncls_b']), False),
        (p1['q_w'], False), (r1(p1['q_b']), False),
        (p1['kv_w'], False), (r2(p1['kv_b']), False),
        (r2(p1['nx_w']), False), (r2(p1['nx_b']), False),
        (p1['proj_w'], False), (r1(p1['proj_b']), False),
        (w9, False), (r2(p1['pp_b']), False),
        (r2(p1['pp_bn_g']), False), (r2(p1['pp_bn_b']), False),
        (r1(p2['ncls_g']), False), (r1(p2['ncls_b']), False),
        (p2['q_w'], False), (r2(p2['q_b']), False),
        (p2['kv_w'], False), (r1(p2['kv_b']), False),
        (r2(p2['nx_w']), False), (r2(p2['nx_b']), False),
    ]
    ka_shapes = [(CLSN, C), (C, N), (1, C), (1, C), (C, C), (1, C),
                 (2 * C, C), (2 * C, 1), (C, 1), (C, 1), (C, C), (1, C),
                 (9, C, CLSN), (C, 1), (C, 1), (C, 1),
                 (1, C), (1, C), (C, C), (C, 1), (2 * C, C), (1, 2 * C),
                 (C, 1), (C, 1)]
    out_cls, xp2, attn_pre, vv = _call(
        _ka_body, batch, ka_ins, ka_shapes,
        [(CLSN, C), (C, N), (G, N), (CLSN, C)])

    kb_ins = [
        (attn_pre, True),
        (pg['fc1_w'], False), (r2(pg['fc1_b']), False),
        (r2(pg['fc1_bn_g']), False), (r2(pg['fc1_bn_b']), False),
    ]
    kb_shapes = [(G, N), (G, G), (G, 1), (G, 1), (G, 1)]
    x1b, fpad, idx = _call(_kb_body, batch, kb_ins, kb_shapes,
                           [(G, N), (N, DPAD), (N, KNN)],
                           [jnp.float32, jnp.float32, jnp.int32])

    table = fpad.reshape(batch * N, DPAD)
    npw = batch * N // NWORK
    idx_sc = (idx.transpose(2, 0, 1).reshape(KNN, NWORK, 2, npw // 2)
              .transpose(1, 2, 0, 3).reshape(NWORK, KNN * npw))
    maxnt = _sc_gather_max(table, idx_sc).reshape(batch, N, DPAD)

    wf = pg['nn_w'][:, :, 0::2]
    wm = pg['nn_w'][:, :, 1::2]
    wfm = wf - wm
    gpg = 2 * G // NH
    kc_ins = [
        (attn_pre, True), (x1b, True), (maxnt, True),
        (vv, True), (out_cls, True), (xp2, True),
        (wfm, False), (wm, False),
        (r2(pg['nn_b']), False), (r2(pg['nn_bn_g']), False),
        (r2(pg['nn_bn_b']), False),
        (pg['fc2_w'], False), (r2(pg['fc2_b']), False),
        (r2(pg['fc2_bn_g']), False), (r2(pg['fc2_bn_b']), False),
        (p2['proj_w'], False), (r2(p2['proj_b']), False),
        (r1(params['norm_g']), False), (r1(params['norm_b']), False),
        (params['mlp_fc1_w'], False), (r1(params['mlp_fc1_b']), False),
        (params['mlp_fc2_w'], False), (r1(params['mlp_fc2_b']), False),
    ]
    kc_shapes = [(G, N), (G, N), (N, DPAD), (CLSN, C), (CLSN, C), (C, N),
                 (NH, gpg, CLSN), (NH, gpg, CLSN),
                 (2 * G, 1), (2 * G, 1), (2 * G, 1),
                 (G, 2 * G), (G, 1), (G, 1), (G, 1),
                 (C, C), (C, 1), (1, C), (1, C),
                 (4 * C, C), (1, 4 * C), (C, 4 * C), (1, C)]
    cls_out, patch_out = _call(_kc_body, batch, kc_ins, kc_shapes,
                               [(CLSN, C), (C, N)])
    return cls_out, patch_out.reshape(batch, C, HP, WP)


# speedup vs baseline: 1.1533x; 1.0120x over previous
import functools

import jax
import jax.numpy as jnp
from jax.experimental import pallas as pl
from jax.experimental.pallas import tpu as pltpu
from jax.experimental.pallas import tpu_sc as plsc

C = 384
CLSN = 150
NH = 4
HD = C // NH
HP = 32
WP = 32
N = HP * WP
KNN = 9
G = NH * CLSN
EPS = 1e-5
SCALE = HD ** -0.5
DPAD = 640
NWORK = 32


def _ln_rows(x, g, b):
    m = jnp.mean(x, axis=1, keepdims=True)
    m2 = jnp.mean(x * x, axis=1, keepdims=True)
    v = m2 - m * m
    return (x - m) * jax.lax.rsqrt(v + EPS) * g + b


def _lng_block(x, w, b):
    m = jnp.mean(x)
    v = jnp.mean(x * x) - m * m
    return (x - m) * jax.lax.rsqrt(v + EPS) * w + b


def _dot(a, b):
    return jax.lax.dot_general(a, b, (((1,), (0,)), ((), ())),
                               preferred_element_type=jnp.float32)


def _dot_tb(a, b):
    return jax.lax.dot_general(a, b, (((1,), (1,)), ((), ())),
                               preferred_element_type=jnp.float32)


def _gelu(x):
    return jax.nn.gelu(x, approximate=True)


def _ka_body(xc_ref, xp_ref, ncls_g, ncls_b, qw, qb, kvw, kvb, nxw, nxb,
             projw, projb, w9, ppb, ppg, ppbb,
             ncls_g2, ncls_b2, qw2, qb2, kvw2, kvb2, nxw2, nxb2,
             out_cls_ref, xp2_ref, attn_ref, vv_ref):
    xc = xc_ref[0]
    xp = xp_ref[0]
    xl = _ln_rows(xc, ncls_g[...], ncls_b[...])
    q = _dot_tb(xl, qw[...]) + qb[...]
    xn = _lng_block(xp, nxw[...], nxb[...])
    kv = _dot(kvw[...], xn) + kvb[...]
    outs = []
    for h in range(NH):
        qh = q[:, h * HD:(h + 1) * HD]
        kh = kv[h * HD:(h + 1) * HD, :]
        vh = kv[C + h * HD:C + (h + 1) * HD, :]
        lg = _dot(qh, kh) * SCALE
        lg = lg - jnp.max(lg, axis=1, keepdims=True)
        e = jnp.exp(lg)
        p = e / jnp.sum(e, axis=1, keepdims=True)
        outs.append(_dot_tb(p, vh))
    oc = jnp.concatenate(outs, axis=1)
    out_cls = xc + _dot_tb(oc, projw[...]) + projb[...]
    out_cls_ref[0] = out_cls

    op = _dot(out_cls, xp)
    col = jax.lax.broadcasted_iota(jnp.int32, (1, N), 1) % WP
    row = jax.lax.broadcasted_iota(jnp.int32, (1, N), 1) // WP
    acc = jnp.zeros((C, N), jnp.float32)
    for ky in range(3):
        for kx in range(3):
            off = (ky - 1) * WP + (kx - 1)
            if off > 0:
                sh = jnp.concatenate(
                    [op[:, off:], jnp.zeros((CLSN, off), jnp.float32)], axis=1)
            elif off < 0:
                sh = jnp.concatenate(
                    [jnp.zeros((CLSN, -off), jnp.float32), op[:, :N + off]],
                    axis=1)
            else:
                sh = op
            mask = ((col + (kx - 1) >= 0) & (col + (kx - 1) < WP) &
                    (row + (ky - 1) >= 0) & (row + (ky - 1) < HP))
            sh = jnp.where(mask, sh, 0.0)
            acc = acc + _dot(w9[3 * ky + kx], sh)
    op2 = (acc + ppb[...]) * ppg[...] + ppbb[...]
    xp2 = xp + _gelu(op2)
    xp2_ref[0] = xp2

    xn2 = _lng_block(xp2, nxw2[...], nxb2[...])
    q2 = _dot(qw2[...], xn2) + qb2[...]
    clsn = _ln_rows(out_cls, ncls_g2[...], ncls_b2[...])
    kv2 = _dot_tb(clsn, kvw2[...]) + kvb2[...]
    kk = kv2[:, :C]
    vv_ref[0] = kv2[:, C:]
    blocks = []
    for h in range(NH):
        kh = kk[:, h * HD:(h + 1) * HD]
        qh = q2[h * HD:(h + 1) * HD, :]
        blocks.append(_dot(kh, qh) * SCALE)
    attn_ref[0] = jnp.concatenate(blocks, axis=0)


def _kb_body(x_ref, fc1w, fc1b, fc1g, fc1bb, x1_ref, fpad_ref, idx_ref):
    x = x_ref[0]
    x1 = _dot(fc1w[...], x) + fc1b[...]
    x1 = x1 * fc1g[...] + fc1bb[...]
    x1_ref[0] = x1

    f = x1.T
    fpad_ref[0, :, :G] = f
    fpad_ref[0, :, G:] = jnp.zeros((N, DPAD - G), jnp.float32)

    gram = _dot_tb(f, f)
    sq_row = jnp.sum(x1 * x1, axis=0, keepdims=True)
    dist = sq_row - 2.0 * gram

    gbase = pl.program_id(0) * N
    iotaf = jax.lax.broadcasted_iota(jnp.int32, (N, N), 1).astype(jnp.float32)
    for k in range(KNN):
        vmin = jnp.min(dist, axis=1, keepdims=True)
        idxf = jnp.min(jnp.where(dist <= vmin, iotaf, jnp.float32(2.0 * N)),
                       axis=1, keepdims=True)
        idx_ref[0, :, k:k + 1] = idxf.astype(jnp.int32) + gbase
        if k < KNN - 1:
            dist = jnp.where(iotaf == idxf, jnp.inf, dist)


def _sc_gather_max(table, idx):
    rows, d = table.shape
    npw = rows // NWORK
    mesh = plsc.VectorSubcoreMesh(core_axis_name="c", subcore_axis_name="s")

    nvr = (G + 15) // 16
    half = npw // 2
    hidx = KNN * half

    @functools.partial(
        pl.kernel, mesh=mesh,
        out_type=jax.ShapeDtypeStruct((rows, d), jnp.float32),
        scratch_types=[
            pltpu.VMEM((2 * hidx,), jnp.int32),
            pltpu.VMEM((half, d), jnp.float32),
            pltpu.VMEM((half, d), jnp.float32),
            pltpu.VMEM((half, d), jnp.float32),
            pltpu.VMEM((half, d), jnp.float32),
            pltpu.VMEM((half, d), jnp.float32),
            pltpu.SemaphoreType.DMA,
            pltpu.SemaphoreType.DMA,
            pltpu.SemaphoreType.DMA,
            pltpu.SemaphoreType.DMA,
            pltpu.SemaphoreType.DMA,
        ],
    )
    def run(table_hbm, idx_hbm, out_hbm, idxv, accv, r0, r1, r2, r3,
            sema, s0, s1, s2, s3):
        wid = jax.lax.axis_index("s") * 2 + jax.lax.axis_index("c")
        rbuf = (r0, r1, r2, r3)
        rsem = (s0, s1, s2, s3)
        pltpu.sync_copy(idx_hbm.at[wid], idxv)

        def gather(h, k, dst, sem):
            isl = idxv.at[pl.ds(h * hidx + k * half, half)]
            return pltpu.async_copy(table_hbm.at[isl], dst, sem)

        for h in range(2):
            cp_acc = gather(h, 0, accv, sema)
            cps = {1: gather(h, 1, r0, s0), 2: gather(h, 2, r1, s1)}
            cp_acc.wait()
            for p in range(4):
                if p < 3:
                    ba, bb = (2 * (p + 1)) % 4, (2 * (p + 1) + 1) % 4
                    cps[2 * p + 3] = gather(h, 2 * p + 3, rbuf[ba], rsem[ba])
                    cps[2 * p + 4] = gather(h, 2 * p + 4, rbuf[bb], rsem[bb])
                cps[2 * p + 1].wait()
                cps[2 * p + 2].wait()
                ra = rbuf[(2 * p) % 4]
                rb = rbuf[(2 * p + 1) % 4]

                def body(i, _, _ra=ra, _rb=rb):
                    for j in range(nvr):
                        sl = pl.ds(j * 16, 16)
                        accv[i, sl] = jnp.maximum(
                            accv[i, sl], jnp.maximum(_ra[i, sl], _rb[i, sl]))
                    return 0

                jax.lax.fori_loop(0, half, body, 0)
            pltpu.sync_copy(accv,
                            out_hbm.at[pl.ds(wid * npw + h * half, half)])

    return run(table, idx)


def _kc_body(x_ref, x1_ref, mt_ref, vv_ref, cls_ref, xp2_ref,
             wfm, wm, nnb, nng, nnbb, fc2w, fc2b, fc2g, fc2bb,
             projw, projb, normg, normb, m1w, m1b, m2w, m2b,
             cls_out_ref, patch_out_ref):
    x = x_ref[0]
    x1 = x1_ref[0]
    mt = mt_ref[0]
    ys = []
    for g in range(NH):
        xg = x1[g * CLSN:(g + 1) * CLSN, :]
        mtg = mt[:, g * CLSN:(g + 1) * CLSN]
        ys.append(_dot(wfm[g], xg) + _dot_tb(wm[g], mtg))
    y = jnp.concatenate(ys, axis=0) + nnb[...]
    y = _gelu(y * nng[...] + nnbb[...])
    gout = _dot(fc2w[...], y) + fc2b[...]
    gout = gout * fc2g[...] + fc2bb[...] + x

    vv = vv_ref[0]
    vvt = vv.T
    outs = []
    for h in range(NH):
        blk = gout[h * CLSN:(h + 1) * CLSN, :]
        blk = blk - jnp.max(blk, axis=0, keepdims=True)
        e = jnp.exp(blk)
        p = e / jnp.sum(e, axis=0, keepdims=True)
        vh = vvt[h * HD:(h + 1) * HD, :]
        outs.append(_dot(vh, p))
    o = jnp.concatenate(outs, axis=0)
    patch_out_ref[0] = xp2_ref[0] + _dot(projw[...], o) + projb[...]

    xc = cls_ref[0]
    hl = _ln_rows(xc, normg[...], normb[...])
    h1 = _gelu(_dot_tb(hl, m1w[...]) + m1b[...])
    h2 = _dot_tb(h1, m2w[...]) + m2b[...]
    cls_out_ref[0] = xc + h2


def _bspec(shape):
    nz = (0,) * len(shape)
    return pl.BlockSpec(shape, lambda b, _z=nz: _z)


def _bspecB(shape):
    nz = (0,) * len(shape)
    return pl.BlockSpec((1,) + shape, lambda b, _z=nz: (b,) + _z)


def _call(body, batch, ins, in_shapes, out_shapes, out_dtypes=None):
    in_specs = [(_bspecB(s) if bt else _bspec(s)) for (_, bt), s in
                zip(ins, in_shapes)]
    out_specs = [_bspecB(s) for s in out_shapes]
    if out_dtypes is None:
        out_dtypes = [jnp.float32] * len(out_shapes)
    out_shape = [jax.ShapeDtypeStruct((batch,) + s, dt)
                 for s, dt in zip(out_shapes, out_dtypes)]
    return pl.pallas_call(
        body, grid=(batch,), in_specs=in_specs, out_specs=out_specs,
        out_shape=out_shape,
    )(*[a for a, _ in ins])


def kernel(x_cls, x_patch, params):
    batch = x_cls.shape[0]
    f32 = jnp.float32
    p1 = params['s2f']
    p2 = params['f2s']
    pg = p2['grapher']
    xp = x_patch.reshape(batch, C, N)

    r2 = lambda a: a.reshape(-1, 1).astype(f32)
    r1 = lambda a: a.reshape(1, -1).astype(f32)

    w9 = p1['pp_w'].transpose(2, 3, 0, 1).reshape(9, C, CLSN)
    ka_ins = [
        (x_cls, True), (xp, True),
        (r1(p1['ncls_g']), False), (r1(p1['ncls_b']), False),
        (p1['q_w'], False), (r1(p1['q_b']), False),
        (p1['kv_w'], False), (r2(p1['kv_b']), False),
        (r2(p1['nx_w']), False), (r2(p1['nx_b']), False),
        (p1['proj_w'], False), (r1(p1['proj_b']), False),
        (w9, False), (r2(p1['pp_b']), False),
        (r2(p1['pp_bn_g']), False), (r2(p1['pp_bn_b']), False),
        (r1(p2['ncls_g']), False), (r1(p2['ncls_b']), False),
        (p2['q_w'], False), (r2(p2['q_b']), False),
        (p2['kv_w'], False), (r1(p2['kv_b']), False),
        (r2(p2['nx_w']), False), (r2(p2['nx_b']), False),
    ]
    ka_shapes = [(CLSN, C), (C, N), (1, C), (1, C), (C, C), (1, C),
                 (2 * C, C), (2 * C, 1), (C, 1), (C, 1), (C, C), (1, C),
                 (9, C, CLSN), (C, 1), (C, 1), (C, 1),
                 (1, C), (1, C), (C, C), (C, 1), (2 * C, C), (1, 2 * C),
                 (C, 1), (C, 1)]
    out_cls, xp2, attn_pre, vv = _call(
        _ka_body, batch, ka_ins, ka_shapes,
        [(CLSN, C), (C, N), (G, N), (CLSN, C)])

    kb_ins = [
        (attn_pre, True),
        (pg['fc1_w'], False), (r2(pg['fc1_b']), False),
        (r2(pg['fc1_bn_g']), False), (r2(pg['fc1_bn_b']), False),
    ]
    kb_shapes = [(G, N), (G, G), (G, 1), (G, 1), (G, 1)]
    x1b, fpad, idx = _call(_kb_body, batch, kb_ins, kb_shapes,
                           [(G, N), (N, DPAD), (N, KNN)],
                           [jnp.float32, jnp.float32, jnp.int32])

    table = fpad.reshape(batch * N, DPAD)
    npw = batch * N // NWORK
    idx_sc = (idx.transpose(2, 0, 1).reshape(KNN, NWORK, 2, npw // 2)
              .transpose(1, 2, 0, 3).reshape(NWORK, KNN * npw))
    maxnt = _sc_gather_max(table, idx_sc).reshape(batch, N, DPAD)

    wf = pg['nn_w'][:, :, 0::2]
    wm = pg['nn_w'][:, :, 1::2]
    wfm = wf - wm
    gpg = 2 * G // NH
    kc_ins = [
        (attn_pre, True), (x1b, True), (maxnt, True),
        (vv, True), (out_cls, True), (xp2, True),
        (wfm, False), (wm, False),
        (r2(pg['nn_b']), False), (r2(pg['nn_bn_g']), False),
        (r2(pg['nn_bn_b']), False),
        (pg['fc2_w'], False), (r2(pg['fc2_b']), False),
        (r2(pg['fc2_bn_g']), False), (r2(pg['fc2_bn_b']), False),
        (p2['proj_w'], False), (r2(p2['proj_b']), False),
        (r1(params['norm_g']), False), (r1(params['norm_b']), False),
        (params['mlp_fc1_w'], False), (r1(params['mlp_fc1_b']), False),
        (params['mlp_fc2_w'], False), (r1(params['mlp_fc2_b']), False),
    ]
    kc_shapes = [(G, N), (G, N), (N, DPAD), (CLSN, C), (CLSN, C), (C, N),
                 (NH, gpg, CLSN), (NH, gpg, CLSN),
                 (2 * G, 1), (2 * G, 1), (2 * G, 1),
                 (G, 2 * G), (G, 1), (G, 1), (G, 1),
                 (C, C), (C, 1), (1, C), (1, C),
                 (4 * C, C), (1, 4 * C), (C, 4 * C), (1, C)]
    cls_out, patch_out = _call(_kc_body, batch, kc_ins, kc_shapes,
                               [(CLSN, C), (C, N)])
    return cls_out, patch_out.reshape(batch, C, HP, WP)
